# baseline (speedup 1.0000x reference)
import functools

import jax
import jax.numpy as jnp
from jax import lax
from jax.experimental import pallas as pl
from jax.experimental.pallas import tpu as pltpu
from jax.experimental.pallas import tpu_sc as plsc

VOCAB = 8192
LANES = 128
SUBROWS = VOCAB // LANES


NC = 2
NS = 16
NW = NC * NS
CHUNK = 4


def _sc_gather(n_tokens):
    n_chunks = n_tokens // (NW * CHUNK)
    mesh = plsc.VectorSubcoreMesh(core_axis_name="c", subcore_axis_name="s")

    @functools.partial(
        pl.kernel,
        mesh=mesh,
        out_type=jax.ShapeDtypeStruct(
            (n_tokens // CHUNK, CHUNK, VOCAB), jnp.float32
        ),
        scratch_types=[
            pltpu.VMEM((n_chunks, CHUNK), jnp.int32),
            pltpu.VMEM((CHUNK, VOCAB), jnp.float32),
            pltpu.VMEM((CHUNK, VOCAB), jnp.float32),
            pltpu.SemaphoreType.DMA,
            pltpu.SemaphoreType.DMA,
            pltpu.SemaphoreType.DMA,
            pltpu.SemaphoreType.DMA,
        ],
    )
    def k(x_hbm, table_hbm, out_hbm, idx_v, buf_a, buf_b, ga, gb, oa, ob):
        wid = lax.axis_index("s") * NC + lax.axis_index("c")
        chunk0 = wid * n_chunks
        pltpu.sync_copy(x_hbm.at[pl.ds(chunk0, n_chunks)], idx_v)

        bufs = [buf_a, buf_b]
        gsems = [ga, gb]
        osems = [oa, ob]
        gather = [None, None]
        outcp = [None, None]

        gather[0] = pltpu.async_copy(
            table_hbm.at[idx_v.at[0]], bufs[0], gsems[0]
        )
        for c in range(n_chunks):
            b = c & 1
            gather[b].wait()
            if c >= 1:
                outcp[1 - b].wait()
            if c + 1 < n_chunks:
                gather[1 - b] = pltpu.async_copy(
                    table_hbm.at[idx_v.at[c + 1]], bufs[1 - b], gsems[1 - b]
                )
            outcp[b] = pltpu.async_copy(
                bufs[b], out_hbm.at[chunk0 + c], osems[b]
            )
        outcp[(n_chunks - 1) & 1].wait()

    return k


ROWS_PER_STEP = 16


def _loss_body(x_ref, t_ref, *refs):
    tbl_refs = refs[:ROWS_PER_STEP]
    loss_ref = refs[ROWS_PER_STEP]
    i = pl.program_id(0)
    nsteps = pl.num_programs(0)

    block = jnp.concatenate(
        [tbl_refs[j][...] for j in range(ROWS_PER_STEP)], axis=0
    )

    m = jnp.max(block, axis=(1, 2), keepdims=True)
    s = jnp.sum(jnp.exp(block - m), axis=(1, 2), keepdims=True)

    tv = jnp.stack([t_ref[i * ROWS_PER_STEP + j] for j in range(ROWS_PER_STEP)])
    shape = (ROWS_PER_STEP, SUBROWS, LANES)
    col = (
        jax.lax.broadcasted_iota(jnp.int32, shape, 1) * LANES
        + jax.lax.broadcasted_iota(jnp.int32, shape, 2)
    )
    tgt = jnp.sum(
        jnp.where(col == tv[:, None, None], block, 0.0),
        axis=(1, 2),
        keepdims=True,
    )

    nll_sum = jnp.sum(m + jnp.log(s) - tgt).reshape(1, 1)
    prev = jnp.where(i == 0, jnp.zeros((1, 1), jnp.float32), loss_ref[...])
    tot = prev + nll_sum
    n_tokens = nsteps * ROWS_PER_STEP
    loss_ref[...] = jnp.where(i == nsteps - 1, tot / n_tokens, tot)


def _tc_loss(xf, tf, tbl3):
    n = xf.shape[0]
    nsteps = n // ROWS_PER_STEP
    grid_spec = pltpu.PrefetchScalarGridSpec(
        num_scalar_prefetch=2,
        grid=(nsteps,),
        in_specs=[
            pl.BlockSpec(
                (1, SUBROWS, LANES),
                lambda i, xr, tr, j=j: (xr[i * ROWS_PER_STEP + j], 0, 0),
            )
            for j in range(ROWS_PER_STEP)
        ],
        out_specs=[pl.BlockSpec((1, 1), lambda i, xr, tr: (0, 0))],
    )
    (loss2d,) = pl.pallas_call(
        _loss_body,
        grid_spec=grid_spec,
        out_shape=[jax.ShapeDtypeStruct((1, 1), jnp.float32)],
        compiler_params=pltpu.CompilerParams(
            dimension_semantics=("arbitrary",),
        ),
    )(xf, tf, *([tbl3] * ROWS_PER_STEP))
    return loss2d[0, 0]


def kernel(x, targets, table):
    B, T = x.shape
    n = B * T
    xf = x.reshape(-1)
    tf = targets.reshape(-1)

    x2d = x.reshape(n // CHUNK, CHUNK)
    logits3d = _sc_gather(n)(x2d, table)
    loss = _tc_loss(xf, tf, table.reshape(VOCAB, SUBROWS, LANES))

    return logits3d.reshape(B, T, VOCAB), loss


# speedup vs baseline: 1.3077x; 1.3077x over previous
import functools

import jax
import jax.numpy as jnp
from jax import lax
from jax.experimental import pallas as pl
from jax.experimental.pallas import tpu as pltpu
from jax.experimental.pallas import tpu_sc as plsc

VOCAB = 8192
LANES = 128
SUBROWS = VOCAB // LANES


NC = 2
NS = 16
NW = NC * NS
CHUNK = 4


def _sc_gather(n_tokens):
    n_chunks = n_tokens // (NW * CHUNK)
    mesh = plsc.VectorSubcoreMesh(core_axis_name="c", subcore_axis_name="s")

    @functools.partial(
        pl.kernel,
        mesh=mesh,
        out_type=jax.ShapeDtypeStruct((n_tokens, VOCAB), jnp.float32),
        scratch_types=[
            pltpu.VMEM((n_chunks, CHUNK), jnp.int32),
            pltpu.VMEM((CHUNK, VOCAB), jnp.float32),
            pltpu.VMEM((CHUNK, VOCAB), jnp.float32),
            pltpu.SemaphoreType.DMA,
            pltpu.SemaphoreType.DMA,
            pltpu.SemaphoreType.DMA,
            pltpu.SemaphoreType.DMA,
        ],
    )
    def k(x_hbm, table_hbm, out_hbm, idx_v, buf_a, buf_b, ga, gb, oa, ob):
        wid = lax.axis_index("s") * NC + lax.axis_index("c")
        chunk0 = wid * n_chunks
        pltpu.sync_copy(x_hbm.at[pl.ds(chunk0, n_chunks)], idx_v)

        bufs = [buf_a, buf_b]
        gsems = [ga, gb]
        osems = [oa, ob]
        gather = [None, None]
        outcp = [None, None]

        gather[0] = pltpu.async_copy(
            table_hbm.at[idx_v.at[0]], bufs[0], gsems[0]
        )
        for c in range(n_chunks):
            b = c & 1
            gather[b].wait()
            if c >= 1:
                outcp[1 - b].wait()
            if c + 1 < n_chunks:
                gather[1 - b] = pltpu.async_copy(
                    table_hbm.at[idx_v.at[c + 1]], bufs[1 - b], gsems[1 - b]
                )
            outcp[b] = pltpu.async_copy(
                bufs[b], out_hbm.at[pl.ds((chunk0 + c) * CHUNK, CHUNK)], osems[b]
            )
        outcp[(n_chunks - 1) & 1].wait()

    return k


ROWS_PER_STEP = 16


def _loss_body(x_ref, t_ref, *refs):
    tbl_refs = refs[:ROWS_PER_STEP]
    loss_ref = refs[ROWS_PER_STEP]
    i = pl.program_id(0)
    nsteps = pl.num_programs(0)

    block = jnp.concatenate(
        [tbl_refs[j][...] for j in range(ROWS_PER_STEP)], axis=0
    )

    m = jnp.max(block, axis=(1, 2), keepdims=True)
    s = jnp.sum(jnp.exp(block - m), axis=(1, 2), keepdims=True)

    tv = jnp.stack([t_ref[i * ROWS_PER_STEP + j] for j in range(ROWS_PER_STEP)])
    shape = (ROWS_PER_STEP, SUBROWS, LANES)
    col = (
        jax.lax.broadcasted_iota(jnp.int32, shape, 1) * LANES
        + jax.lax.broadcasted_iota(jnp.int32, shape, 2)
    )
    tgt = jnp.sum(
        jnp.where(col == tv[:, None, None], block, 0.0),
        axis=(1, 2),
        keepdims=True,
    )

    nll_sum = jnp.sum(m + jnp.log(s) - tgt).reshape(1, 1)
    prev = jnp.where(i == 0, jnp.zeros((1, 1), jnp.float32), loss_ref[...])
    tot = prev + nll_sum
    n_tokens = nsteps * ROWS_PER_STEP
    loss_ref[...] = jnp.where(i == nsteps - 1, tot / n_tokens, tot)


def _tc_loss(xf, tf, tbl3):
    n = xf.shape[0]
    nsteps = n // ROWS_PER_STEP
    grid_spec = pltpu.PrefetchScalarGridSpec(
        num_scalar_prefetch=2,
        grid=(nsteps,),
        in_specs=[
            pl.BlockSpec(
                (1, SUBROWS, LANES),
                lambda i, xr, tr, j=j: (xr[i * ROWS_PER_STEP + j], 0, 0),
            )
            for j in range(ROWS_PER_STEP)
        ],
        out_specs=[pl.BlockSpec((1, 1), lambda i, xr, tr: (0, 0))],
    )
    (loss2d,) = pl.pallas_call(
        _loss_body,
        grid_spec=grid_spec,
        out_shape=[jax.ShapeDtypeStruct((1, 1), jnp.float32)],
        compiler_params=pltpu.CompilerParams(
            dimension_semantics=("arbitrary",),
        ),
    )(xf, tf, *([tbl3] * ROWS_PER_STEP))
    return loss2d[0, 0]


def kernel(x, targets, table):
    B, T = x.shape
    n = B * T
    xf = x.reshape(-1)
    tf = targets.reshape(-1)

    x2d = x.reshape(n // CHUNK, CHUNK)
    logits2d = _sc_gather(n)(x2d, table)
    loss = _tc_loss(xf, tf, table.reshape(VOCAB, SUBROWS, LANES))

    return logits2d.reshape(B, T, VOCAB), loss


# speedup vs baseline: 1.5196x; 1.1620x over previous
import functools

import jax
import jax.numpy as jnp
from jax import lax
from jax.experimental import pallas as pl
from jax.experimental.pallas import tpu as pltpu
from jax.experimental.pallas import tpu_sc as plsc

VOCAB = 8192
LANES = 128
SUBROWS = VOCAB // LANES


NC = 2
NS = 16
NW = NC * NS
CHUNK = 4


def _sc_gather(n_tokens):
    n_chunks = n_tokens // (NW * CHUNK)
    mesh = plsc.VectorSubcoreMesh(core_axis_name="c", subcore_axis_name="s")

    @functools.partial(
        pl.kernel,
        mesh=mesh,
        out_type=jax.ShapeDtypeStruct((n_tokens, VOCAB), jnp.float32),
        scratch_types=[
            pltpu.VMEM((n_chunks, CHUNK), jnp.int32),
            pltpu.VMEM((CHUNK, VOCAB), jnp.float32),
            pltpu.VMEM((CHUNK, VOCAB), jnp.float32),
            pltpu.SemaphoreType.DMA,
            pltpu.SemaphoreType.DMA,
            pltpu.SemaphoreType.DMA,
            pltpu.SemaphoreType.DMA,
        ],
    )
    def k(x_hbm, table_hbm, out_hbm, idx_v, buf_a, buf_b, ga, gb, oa, ob):
        wid = lax.axis_index("s") * NC + lax.axis_index("c")
        chunk0 = wid * n_chunks
        pltpu.sync_copy(x_hbm.at[pl.ds(chunk0, n_chunks)], idx_v)

        bufs = [buf_a, buf_b]
        gsems = [ga, gb]
        osems = [oa, ob]
        gather = [None, None]
        outcp = [None, None]

        gather[0] = pltpu.async_copy(
            table_hbm.at[idx_v.at[0]], bufs[0], gsems[0]
        )
        for c in range(n_chunks):
            b = c & 1
            gather[b].wait()
            if c >= 1:
                outcp[1 - b].wait()
            if c + 1 < n_chunks:
                gather[1 - b] = pltpu.async_copy(
                    table_hbm.at[idx_v.at[c + 1]], bufs[1 - b], gsems[1 - b]
                )
            outcp[b] = pltpu.async_copy(
                bufs[b], out_hbm.at[pl.ds((chunk0 + c) * CHUNK, CHUNK)], osems[b]
            )
        outcp[(n_chunks - 1) & 1].wait()

    return k


ROWS_PER_STEP = 32


def _loss_body(x_ref, t_ref, *refs):
    tbl_refs = refs[:ROWS_PER_STEP]
    loss_ref = refs[ROWS_PER_STEP]
    i = pl.program_id(0)
    nsteps = pl.num_programs(0)

    block = jnp.concatenate(
        [tbl_refs[j][...] for j in range(ROWS_PER_STEP)], axis=0
    )

    m = jnp.max(block, axis=(1, 2), keepdims=True)
    s = jnp.sum(jnp.exp(block - m), axis=(1, 2), keepdims=True)

    tv = jnp.stack([t_ref[i * ROWS_PER_STEP + j] for j in range(ROWS_PER_STEP)])
    shape = (ROWS_PER_STEP, SUBROWS, LANES)
    col = (
        jax.lax.broadcasted_iota(jnp.int32, shape, 1) * LANES
        + jax.lax.broadcasted_iota(jnp.int32, shape, 2)
    )
    tgt = jnp.sum(
        jnp.where(col == tv[:, None, None], block, 0.0),
        axis=(1, 2),
        keepdims=True,
    )

    nll_sum = jnp.sum(m + jnp.log(s) - tgt).reshape(1, 1)
    prev = jnp.where(i == 0, jnp.zeros((1, 1), jnp.float32), loss_ref[...])
    tot = prev + nll_sum
    n_tokens = nsteps * ROWS_PER_STEP
    loss_ref[...] = jnp.where(i == nsteps - 1, tot / n_tokens, tot)


def _tc_loss(xf, tf, tbl3):
    n = xf.shape[0]
    nsteps = n // ROWS_PER_STEP
    grid_spec = pltpu.PrefetchScalarGridSpec(
        num_scalar_prefetch=2,
        grid=(nsteps,),
        in_specs=[
            pl.BlockSpec(
                (1, SUBROWS, LANES),
                lambda i, xr, tr, j=j: (xr[i * ROWS_PER_STEP + j], 0, 0),
            )
            for j in range(ROWS_PER_STEP)
        ],
        out_specs=[pl.BlockSpec((1, 1), lambda i, xr, tr: (0, 0))],
    )
    (loss2d,) = pl.pallas_call(
        _loss_body,
        grid_spec=grid_spec,
        out_shape=[jax.ShapeDtypeStruct((1, 1), jnp.float32)],
        compiler_params=pltpu.CompilerParams(
            dimension_semantics=("arbitrary",),
        ),
    )(xf, tf, *([tbl3] * ROWS_PER_STEP))
    return loss2d[0, 0]


def kernel(x, targets, table):
    B, T = x.shape
    n = B * T
    xf = x.reshape(-1)
    tf = targets.reshape(-1)

    x2d = x.reshape(n // CHUNK, CHUNK)
    logits2d = _sc_gather(n)(x2d, table)
    loss = _tc_loss(xf, tf, table.reshape(VOCAB, SUBROWS, LANES))

    return logits2d.reshape(B, T, VOCAB), loss


# speedup vs baseline: 2.5007x; 1.6456x over previous
import functools

import jax
import jax.numpy as jnp
from jax import lax
from jax.experimental import pallas as pl
from jax.experimental.pallas import tpu as pltpu
from jax.experimental.pallas import tpu_sc as plsc

VOCAB = 8192
LANES = 128
SUBROWS = VOCAB // LANES


NC = 2
NS = 16
NW = NC * NS
CHUNK = 4


def _sc_gather(n_tokens):
    n_chunks = n_tokens // (NW * CHUNK)
    mesh = plsc.VectorSubcoreMesh(core_axis_name="c", subcore_axis_name="s")

    @functools.partial(
        pl.kernel,
        mesh=mesh,
        out_type=jax.ShapeDtypeStruct((n_tokens, VOCAB), jnp.float32),
        scratch_types=[
            pltpu.VMEM((n_chunks, CHUNK), jnp.int32),
            pltpu.VMEM((CHUNK, VOCAB), jnp.float32),
            pltpu.VMEM((CHUNK, VOCAB), jnp.float32),
            pltpu.SemaphoreType.DMA,
            pltpu.SemaphoreType.DMA,
            pltpu.SemaphoreType.DMA,
            pltpu.SemaphoreType.DMA,
        ],
    )
    def k(x_hbm, table_hbm, out_hbm, idx_v, buf_a, buf_b, ga, gb, oa, ob):
        wid = lax.axis_index("s") * NC + lax.axis_index("c")
        chunk0 = wid * n_chunks
        pltpu.sync_copy(x_hbm.at[pl.ds(chunk0, n_chunks)], idx_v)

        bufs = [buf_a, buf_b]
        gsems = [ga, gb]
        osems = [oa, ob]
        gather = [None, None]
        outcp = [None, None]

        gather[0] = pltpu.async_copy(
            table_hbm.at[idx_v.at[0]], bufs[0], gsems[0]
        )
        for c in range(n_chunks):
            b = c & 1
            gather[b].wait()
            if c >= 1:
                outcp[1 - b].wait()
            if c + 1 < n_chunks:
                gather[1 - b] = pltpu.async_copy(
                    table_hbm.at[idx_v.at[c + 1]], bufs[1 - b], gsems[1 - b]
                )
            outcp[b] = pltpu.async_copy(
                bufs[b], out_hbm.at[pl.ds((chunk0 + c) * CHUNK, CHUNK)], osems[b]
            )
        outcp[(n_chunks - 1) & 1].wait()

    return k


ROWS_PER_STEP = 32


def _loss_body(t_ref, logit_ref, loss_ref):
    i = pl.program_id(0)
    nsteps = pl.num_programs(0)

    block = logit_ref[...]
    m = jnp.max(block, axis=1, keepdims=True)
    s = jnp.sum(jnp.exp(block - m), axis=1, keepdims=True)

    tv = jnp.stack([t_ref[i * ROWS_PER_STEP + j] for j in range(ROWS_PER_STEP)])
    col = jax.lax.broadcasted_iota(jnp.int32, (ROWS_PER_STEP, VOCAB), 1)
    tgt = jnp.sum(
        jnp.where(col == tv[:, None], block, 0.0), axis=1, keepdims=True
    )

    nll_sum = jnp.sum(m + jnp.log(s) - tgt).reshape(1, 1)
    prev = jnp.where(i == 0, jnp.zeros((1, 1), jnp.float32), loss_ref[...])
    tot = prev + nll_sum
    n_tokens = nsteps * ROWS_PER_STEP
    loss_ref[...] = jnp.where(i == nsteps - 1, tot / n_tokens, tot)


def _tc_loss(tf, logits2d):
    n = logits2d.shape[0]
    nsteps = n // ROWS_PER_STEP
    grid_spec = pltpu.PrefetchScalarGridSpec(
        num_scalar_prefetch=1,
        grid=(nsteps,),
        in_specs=[
            pl.BlockSpec((ROWS_PER_STEP, VOCAB), lambda i, tr: (i, 0)),
        ],
        out_specs=[pl.BlockSpec((1, 1), lambda i, tr: (0, 0))],
    )
    (loss2d,) = pl.pallas_call(
        _loss_body,
        grid_spec=grid_spec,
        out_shape=[jax.ShapeDtypeStruct((1, 1), jnp.float32)],
        compiler_params=pltpu.CompilerParams(
            dimension_semantics=("arbitrary",),
        ),
    )(tf, logits2d)
    return loss2d[0, 0]


def kernel(x, targets, table):
    B, T = x.shape
    n = B * T
    tf = targets.reshape(-1)

    x2d = x.reshape(n // CHUNK, CHUNK)
    logits2d = _sc_gather(n)(x2d, table)
    loss = _tc_loss(tf, logits2d)

    return logits2d.reshape(B, T, VOCAB), loss


# speedup vs baseline: 3.0410x; 1.2161x over previous
import functools

import jax
import jax.numpy as jnp
from jax import lax
from jax.experimental import pallas as pl
from jax.experimental.pallas import tpu as pltpu
from jax.experimental.pallas import tpu_sc as plsc

VOCAB = 8192
LANES = 128
SUBROWS = VOCAB // LANES


NC = 2
NS = 16
NW = NC * NS
CHUNK = 4


def _sc_gather(n_tokens):
    n_chunks = n_tokens // (NW * CHUNK)
    mesh = plsc.VectorSubcoreMesh(core_axis_name="c", subcore_axis_name="s")

    @functools.partial(
        pl.kernel,
        mesh=mesh,
        out_type=jax.ShapeDtypeStruct((n_tokens, VOCAB), jnp.float32),
        scratch_types=[
            pltpu.VMEM((n_chunks, CHUNK), jnp.int32),
            pltpu.VMEM((CHUNK, VOCAB), jnp.float32),
            pltpu.VMEM((CHUNK, VOCAB), jnp.float32),
            pltpu.SemaphoreType.DMA,
            pltpu.SemaphoreType.DMA,
            pltpu.SemaphoreType.DMA,
            pltpu.SemaphoreType.DMA,
        ],
    )
    def k(x_hbm, table_hbm, out_hbm, idx_v, buf_a, buf_b, ga, gb, oa, ob):
        wid = lax.axis_index("s") * NC + lax.axis_index("c")
        chunk0 = wid * n_chunks
        pltpu.sync_copy(x_hbm.at[pl.ds(chunk0, n_chunks)], idx_v)

        bufs = [buf_a, buf_b]
        gsems = [ga, gb]
        osems = [oa, ob]
        gather = [None, None]
        outcp = [None, None]

        gather[0] = pltpu.async_copy(
            table_hbm.at[idx_v.at[0]], bufs[0], gsems[0]
        )
        for c in range(n_chunks):
            b = c & 1
            gather[b].wait()
            if c >= 1:
                outcp[1 - b].wait()
            if c + 1 < n_chunks:
                gather[1 - b] = pltpu.async_copy(
                    table_hbm.at[idx_v.at[c + 1]], bufs[1 - b], gsems[1 - b]
                )
            outcp[b] = pltpu.async_copy(
                bufs[b], out_hbm.at[pl.ds((chunk0 + c) * CHUNK, CHUNK)], osems[b]
            )
        outcp[(n_chunks - 1) & 1].wait()

    return k


ROWS_PER_STEP = 64


def _loss_body(t_ref, logit_ref, loss_ref):
    i = pl.program_id(0)
    nsteps = pl.num_programs(0)

    block = logit_ref[...]
    m = jnp.max(block, axis=1, keepdims=True)
    s = jnp.sum(jnp.exp(block - m), axis=1, keepdims=True)

    tv = jnp.stack([t_ref[i * ROWS_PER_STEP + j] for j in range(ROWS_PER_STEP)])
    col = jax.lax.broadcasted_iota(jnp.int32, (ROWS_PER_STEP, VOCAB), 1)
    tgt = jnp.sum(
        jnp.where(col == tv[:, None], block, 0.0), axis=1, keepdims=True
    )

    nll_sum = jnp.sum(m + jnp.log(s) - tgt).reshape(1, 1)
    prev = jnp.where(i == 0, jnp.zeros((1, 1), jnp.float32), loss_ref[...])
    tot = prev + nll_sum
    n_tokens = nsteps * ROWS_PER_STEP
    loss_ref[...] = jnp.where(i == nsteps - 1, tot / n_tokens, tot)


def _tc_loss(tf, logits2d):
    n = logits2d.shape[0]
    nsteps = n // ROWS_PER_STEP
    grid_spec = pltpu.PrefetchScalarGridSpec(
        num_scalar_prefetch=1,
        grid=(nsteps,),
        in_specs=[
            pl.BlockSpec((ROWS_PER_STEP, VOCAB), lambda i, tr: (i, 0)),
        ],
        out_specs=[pl.BlockSpec((1, 1), lambda i, tr: (0, 0))],
    )
    (loss2d,) = pl.pallas_call(
        _loss_body,
        grid_spec=grid_spec,
        out_shape=[jax.ShapeDtypeStruct((1, 1), jnp.float32)],
        compiler_params=pltpu.CompilerParams(
            dimension_semantics=("arbitrary",),
        ),
    )(tf, logits2d)
    return loss2d[0, 0]


def kernel(x, targets, table):
    B, T = x.shape
    n = B * T
    tf = targets.reshape(-1)

    x2d = x.reshape(n // CHUNK, CHUNK)
    logits2d = _sc_gather(n)(x2d, table)
    loss = _tc_loss(tf, logits2d)

    return logits2d.reshape(B, T, VOCAB), loss


# speedup vs baseline: 3.3416x; 1.0989x over previous
import functools

import jax
import jax.numpy as jnp
from jax import lax
from jax.experimental import pallas as pl
from jax.experimental.pallas import tpu as pltpu
from jax.experimental.pallas import tpu_sc as plsc

VOCAB = 8192
LANES = 128
SUBROWS = VOCAB // LANES


NC = 2
NS = 16
NW = NC * NS
CHUNK = 4


def _sc_gather(n_tokens):
    n_chunks = n_tokens // (NW * CHUNK)
    mesh = plsc.VectorSubcoreMesh(core_axis_name="c", subcore_axis_name="s")

    @functools.partial(
        pl.kernel,
        mesh=mesh,
        out_type=jax.ShapeDtypeStruct((n_tokens, VOCAB), jnp.float32),
        scratch_types=[
            pltpu.VMEM((n_chunks, CHUNK), jnp.int32),
            pltpu.VMEM((CHUNK, VOCAB), jnp.float32),
            pltpu.VMEM((CHUNK, VOCAB), jnp.float32),
            pltpu.SemaphoreType.DMA,
            pltpu.SemaphoreType.DMA,
            pltpu.SemaphoreType.DMA,
            pltpu.SemaphoreType.DMA,
        ],
    )
    def k(x_hbm, table_hbm, out_hbm, idx_v, buf_a, buf_b, ga, gb, oa, ob):
        wid = lax.axis_index("s") * NC + lax.axis_index("c")
        chunk0 = wid * n_chunks
        pltpu.sync_copy(x_hbm.at[pl.ds(chunk0, n_chunks)], idx_v)

        bufs = [buf_a, buf_b]
        gsems = [ga, gb]
        osems = [oa, ob]
        gather = [None, None]
        outcp = [None, None]

        gather[0] = pltpu.async_copy(
            table_hbm.at[idx_v.at[0]], bufs[0], gsems[0]
        )
        for c in range(n_chunks):
            b = c & 1
            gather[b].wait()
            if c >= 1:
                outcp[1 - b].wait()
            if c + 1 < n_chunks:
                gather[1 - b] = pltpu.async_copy(
                    table_hbm.at[idx_v.at[c + 1]], bufs[1 - b], gsems[1 - b]
                )
            outcp[b] = pltpu.async_copy(
                bufs[b], out_hbm.at[pl.ds((chunk0 + c) * CHUNK, CHUNK)], osems[b]
            )
        outcp[(n_chunks - 1) & 1].wait()

    return k


ROWS_PER_STEP = 128


def _loss_body(t_ref, logit_ref, loss_ref):
    i = pl.program_id(0)
    nsteps = pl.num_programs(0)

    block = logit_ref[...]
    m = jnp.max(block, axis=1, keepdims=True)
    s = jnp.sum(jnp.exp(block - m), axis=1, keepdims=True)

    tv = jnp.stack([t_ref[i * ROWS_PER_STEP + j] for j in range(ROWS_PER_STEP)])
    col = jax.lax.broadcasted_iota(jnp.int32, (ROWS_PER_STEP, VOCAB), 1)
    tgt = jnp.sum(
        jnp.where(col == tv[:, None], block, 0.0), axis=1, keepdims=True
    )

    nll_sum = jnp.sum(m + jnp.log(s) - tgt).reshape(1, 1)
    prev = jnp.where(i == 0, jnp.zeros((1, 1), jnp.float32), loss_ref[...])
    tot = prev + nll_sum
    n_tokens = nsteps * ROWS_PER_STEP
    loss_ref[...] = jnp.where(i == nsteps - 1, tot / n_tokens, tot)


def _tc_loss(tf, logits2d):
    n = logits2d.shape[0]
    nsteps = n // ROWS_PER_STEP
    grid_spec = pltpu.PrefetchScalarGridSpec(
        num_scalar_prefetch=1,
        grid=(nsteps,),
        in_specs=[
            pl.BlockSpec((ROWS_PER_STEP, VOCAB), lambda i, tr: (i, 0)),
        ],
        out_specs=[pl.BlockSpec((1, 1), lambda i, tr: (0, 0))],
    )
    (loss2d,) = pl.pallas_call(
        _loss_body,
        grid_spec=grid_spec,
        out_shape=[jax.ShapeDtypeStruct((1, 1), jnp.float32)],
        compiler_params=pltpu.CompilerParams(
            dimension_semantics=("arbitrary",),
        ),
    )(tf, logits2d)
    return loss2d[0, 0]


def kernel(x, targets, table):
    B, T = x.shape
    n = B * T
    tf = targets.reshape(-1)

    x2d = x.reshape(n // CHUNK, CHUNK)
    logits2d = _sc_gather(n)(x2d, table)
    loss = _tc_loss(tf, logits2d)

    return logits2d.reshape(B, T, VOCAB), loss


# speedup vs baseline: 3.5278x; 1.0557x over previous
import functools

import jax
import jax.numpy as jnp
from jax import lax
from jax.experimental import pallas as pl
from jax.experimental.pallas import tpu as pltpu
from jax.experimental.pallas import tpu_sc as plsc

VOCAB = 8192
LANES = 128
SUBROWS = VOCAB // LANES


NC = 2
NS = 16
NW = NC * NS
CHUNK = 4


def _sc_gather(n_tokens):
    n_chunks = n_tokens // (NW * CHUNK)
    mesh = plsc.VectorSubcoreMesh(core_axis_name="c", subcore_axis_name="s")

    @functools.partial(
        pl.kernel,
        mesh=mesh,
        out_type=jax.ShapeDtypeStruct((n_tokens, VOCAB), jnp.float32),
        scratch_types=[
            pltpu.VMEM((n_chunks, CHUNK), jnp.int32),
            pltpu.VMEM((CHUNK, VOCAB), jnp.float32),
            pltpu.VMEM((CHUNK, VOCAB), jnp.float32),
            pltpu.SemaphoreType.DMA,
            pltpu.SemaphoreType.DMA,
            pltpu.SemaphoreType.DMA,
            pltpu.SemaphoreType.DMA,
        ],
    )
    def k(x_hbm, table_hbm, out_hbm, idx_v, buf_a, buf_b, ga, gb, oa, ob):
        wid = lax.axis_index("s") * NC + lax.axis_index("c")
        chunk0 = wid * n_chunks
        pltpu.sync_copy(x_hbm.at[pl.ds(chunk0, n_chunks)], idx_v)

        bufs = [buf_a, buf_b]
        gsems = [ga, gb]
        osems = [oa, ob]
        gather = [None, None]
        outcp = [None, None]

        gather[0] = pltpu.async_copy(
            table_hbm.at[idx_v.at[0]], bufs[0], gsems[0]
        )
        for c in range(n_chunks):
            b = c & 1
            gather[b].wait()
            if c >= 1:
                outcp[1 - b].wait()
            if c + 1 < n_chunks:
                gather[1 - b] = pltpu.async_copy(
                    table_hbm.at[idx_v.at[c + 1]], bufs[1 - b], gsems[1 - b]
                )
            outcp[b] = pltpu.async_copy(
                bufs[b], out_hbm.at[pl.ds((chunk0 + c) * CHUNK, CHUNK)], osems[b]
            )
        outcp[(n_chunks - 1) & 1].wait()

    return k


ROWS_PER_STEP = 256


def _loss_body(t_ref, logit_ref, loss_ref):
    i = pl.program_id(0)
    nsteps = pl.num_programs(0)

    block = logit_ref[...]
    m = jnp.max(block, axis=1, keepdims=True)
    s = jnp.sum(jnp.exp(block - m), axis=1, keepdims=True)

    tv = jnp.stack([t_ref[i * ROWS_PER_STEP + j] for j in range(ROWS_PER_STEP)])
    col = jax.lax.broadcasted_iota(jnp.int32, (ROWS_PER_STEP, VOCAB), 1)
    tgt = jnp.sum(
        jnp.where(col == tv[:, None], block, 0.0), axis=1, keepdims=True
    )

    nll_sum = jnp.sum(m + jnp.log(s) - tgt).reshape(1, 1)
    prev = jnp.where(i == 0, jnp.zeros((1, 1), jnp.float32), loss_ref[...])
    tot = prev + nll_sum
    n_tokens = nsteps * ROWS_PER_STEP
    loss_ref[...] = jnp.where(i == nsteps - 1, tot / n_tokens, tot)


def _tc_loss(tf, logits2d):
    n = logits2d.shape[0]
    nsteps = n // ROWS_PER_STEP
    grid_spec = pltpu.PrefetchScalarGridSpec(
        num_scalar_prefetch=1,
        grid=(nsteps,),
        in_specs=[
            pl.BlockSpec((ROWS_PER_STEP, VOCAB), lambda i, tr: (i, 0)),
        ],
        out_specs=[pl.BlockSpec((1, 1), lambda i, tr: (0, 0))],
    )
    (loss2d,) = pl.pallas_call(
        _loss_body,
        grid_spec=grid_spec,
        out_shape=[jax.ShapeDtypeStruct((1, 1), jnp.float32)],
        compiler_params=pltpu.CompilerParams(
            dimension_semantics=("arbitrary",),
        ),
    )(tf, logits2d)
    return loss2d[0, 0]


def kernel(x, targets, table):
    B, T = x.shape
    n = B * T
    tf = targets.reshape(-1)

    x2d = x.reshape(n // CHUNK, CHUNK)
    logits2d = _sc_gather(n)(x2d, table)
    loss = _tc_loss(tf, logits2d)

    return logits2d.reshape(B, T, VOCAB), loss


# speedup vs baseline: 3.5749x; 1.0134x over previous
import functools

import jax
import jax.numpy as jnp
from jax import lax
from jax.experimental import pallas as pl
from jax.experimental.pallas import tpu as pltpu
from jax.experimental.pallas import tpu_sc as plsc

VOCAB = 8192
LANES = 128
SUBROWS = VOCAB // LANES


NC = 2
NS = 16
NW = NC * NS
CHUNK = 4


def _sc_gather(n_tokens):
    n_chunks = n_tokens // (NW * CHUNK)
    mesh = plsc.VectorSubcoreMesh(core_axis_name="c", subcore_axis_name="s")

    @functools.partial(
        pl.kernel,
        mesh=mesh,
        out_type=jax.ShapeDtypeStruct((n_tokens, VOCAB), jnp.float32),
        scratch_types=[
            pltpu.VMEM((n_chunks, CHUNK), jnp.int32),
            pltpu.VMEM((CHUNK, VOCAB), jnp.float32),
            pltpu.VMEM((CHUNK, VOCAB), jnp.float32),
            pltpu.SemaphoreType.DMA,
            pltpu.SemaphoreType.DMA,
            pltpu.SemaphoreType.DMA,
            pltpu.SemaphoreType.DMA,
        ],
    )
    def k(x_hbm, table_hbm, out_hbm, idx_v, buf_a, buf_b, ga, gb, oa, ob):
        wid = lax.axis_index("s") * NC + lax.axis_index("c")
        chunk0 = wid * n_chunks
        pltpu.sync_copy(x_hbm.at[pl.ds(chunk0, n_chunks)], idx_v)

        bufs = [buf_a, buf_b]
        gsems = [ga, gb]
        osems = [oa, ob]
        gather = [None, None]
        outcp = [None, None]

        gather[0] = pltpu.async_copy(
            table_hbm.at[idx_v.at[0]], bufs[0], gsems[0]
        )
        for c in range(n_chunks):
            b = c & 1
            gather[b].wait()
            if c >= 1:
                outcp[1 - b].wait()
            if c + 1 < n_chunks:
                gather[1 - b] = pltpu.async_copy(
                    table_hbm.at[idx_v.at[c + 1]], bufs[1 - b], gsems[1 - b]
                )
            outcp[b] = pltpu.async_copy(
                bufs[b], out_hbm.at[pl.ds((chunk0 + c) * CHUNK, CHUNK)], osems[b]
            )
        outcp[(n_chunks - 1) & 1].wait()

    return k


ROWS_PER_STEP = 512


def _loss_body(t_ref, logit_ref, loss_ref):
    i = pl.program_id(0)
    nsteps = pl.num_programs(0)

    block = logit_ref[...]
    m = jnp.max(block, axis=1, keepdims=True)
    s = jnp.sum(jnp.exp(block - m), axis=1, keepdims=True)

    tv = jnp.stack([t_ref[i * ROWS_PER_STEP + j] for j in range(ROWS_PER_STEP)])
    col = jax.lax.broadcasted_iota(jnp.int32, (ROWS_PER_STEP, VOCAB), 1)
    tgt = jnp.sum(
        jnp.where(col == tv[:, None], block, 0.0), axis=1, keepdims=True
    )

    nll_sum = jnp.sum(m + jnp.log(s) - tgt).reshape(1, 1)
    prev = jnp.where(i == 0, jnp.zeros((1, 1), jnp.float32), loss_ref[...])
    tot = prev + nll_sum
    n_tokens = nsteps * ROWS_PER_STEP
    loss_ref[...] = jnp.where(i == nsteps - 1, tot / n_tokens, tot)


def _tc_loss(tf, logits2d):
    n = logits2d.shape[0]
    nsteps = n // ROWS_PER_STEP
    grid_spec = pltpu.PrefetchScalarGridSpec(
        num_scalar_prefetch=1,
        grid=(nsteps,),
        in_specs=[
            pl.BlockSpec((ROWS_PER_STEP, VOCAB), lambda i, tr: (i, 0)),
        ],
        out_specs=[pl.BlockSpec((1, 1), lambda i, tr: (0, 0))],
    )
    (loss2d,) = pl.pallas_call(
        _loss_body,
        grid_spec=grid_spec,
        out_shape=[jax.ShapeDtypeStruct((1, 1), jnp.float32)],
        compiler_params=pltpu.CompilerParams(
            dimension_semantics=("arbitrary",),
        ),
    )(tf, logits2d)
    return loss2d[0, 0]


def kernel(x, targets, table):
    B, T = x.shape
    n = B * T
    tf = targets.reshape(-1)

    x2d = x.reshape(n // CHUNK, CHUNK)
    logits2d = _sc_gather(n)(x2d, table)
    loss = _tc_loss(tf, logits2d)

    return logits2d.reshape(B, T, VOCAB), loss
